# baseline (speedup 1.0000x reference)
import functools

import jax
import jax.numpy as jnp
from jax import lax
from jax.experimental import pallas as pl
from jax.experimental.pallas import tpu as pltpu
from jax.experimental.pallas import tpu_sc as plsc

EMBED_DIM = 32
BATCH = 16384
LANES = 16

NUM_CORES = 2
NUM_SUBCORES = 16
NW = NUM_CORES * NUM_SUBCORES
B_PER_W = BATCH // NW
RB = 1
NROUND = B_PER_W // RB
NBUF = 16

_mesh = plsc.VectorSubcoreMesh(core_axis_name="c", subcore_axis_name="s")


@functools.partial(
    pl.kernel,
    mesh=_mesh,
    out_type=jax.ShapeDtypeStruct((EMBED_DIM, BATCH), jnp.float32),
    scratch_types=[
        pltpu.VMEM_SHARED((NUM_SUBCORES, B_PER_W), jnp.int32),
        pltpu.SMEM((B_PER_W,), jnp.int32),
        pltpu.VMEM((NBUF, RB, EMBED_DIM, 128), jnp.float32),
        pltpu.VMEM((EMBED_DIM, B_PER_W), jnp.float32),
        [pltpu.SemaphoreType.DMA] * NBUF,
    ],
    compiler_params=pltpu.CompilerParams(
        use_tc_tiling_on_sc=True, needs_layout_passes=False
    ),
)
def _gather_kernel(tv_hbm, idx_hbm, out_hbm, idx_sh, idx_s, blk_v, stage_v,
                   sems):
    sid = lax.axis_index("s")
    wid = sid * NUM_CORES + lax.axis_index("c")
    base = wid * B_PER_W
    pltpu.sync_copy(idx_hbm.at[pl.ds(base, B_PER_W)], idx_sh.at[sid])
    pltpu.sync_copy(idx_sh.at[sid], idx_s)

    d_lo = lax.iota(jnp.int32, LANES)
    d_hi = d_lo + LANES

    def fire(r, buf):
        k0 = r * RB
        for t in range(RB):
            i = idx_s[k0 + t]
            blk = pl.multiple_of((i // 128) * 128, 128)
            pltpu.async_copy(
                tv_hbm.at[:, pl.ds(blk, 128)], blk_v.at[buf, t], sems[buf]
            )

    def drain_extract(r, buf):
        for t in range(RB):
            pltpu.make_async_copy(
                tv_hbm.at[:, pl.ds(0, 128)], blk_v.at[buf, t], sems[buf]
            ).wait()
        k0 = r * RB
        for t in range(RB):
            i = idx_s[k0 + t]
            j = lax.rem(i, 128)
            col = jnp.full((LANES,), j, jnp.int32)
            row_k = jnp.full((LANES,), k0 + t, jnp.int32)
            v_lo = plsc.load_gather(blk_v.at[buf, t], [d_lo, col])
            v_hi = plsc.load_gather(blk_v.at[buf, t], [d_hi, col])
            plsc.store_scatter(stage_v, [d_lo, row_k], v_lo)
            plsc.store_scatter(stage_v, [d_hi, row_k], v_hi)

    for q in range(NBUF - 1):
        fire(q, q)

    def group_body(p, c):
        r0 = p * NBUF
        for q in range(NBUF):
            r = r0 + q
            fb = (q + NBUF - 1) % NBUF

            @pl.when(r + NBUF - 1 < NROUND)
            def _():
                fire(r + NBUF - 1, fb)

            drain_extract(r, q)
        return c

    lax.fori_loop(0, NROUND // NBUF, group_body, 0)
    pltpu.sync_copy(stage_v, out_hbm.at[:, pl.ds(base, B_PER_W)])


def kernel(broadcaster, table):
    idx = broadcaster.astype(jnp.int32)
    out_t = _gather_kernel(table.T, idx)
    return out_t.T


# speedup vs baseline: 1.0205x; 1.0205x over previous
import functools

import jax
import jax.numpy as jnp
from jax import lax
from jax.experimental import pallas as pl
from jax.experimental.pallas import tpu as pltpu
from jax.experimental.pallas import tpu_sc as plsc

EMBED_DIM = 32
BATCH = 16384
LANES = 16

NUM_CORES = 2
NUM_SUBCORES = 16
NW = NUM_CORES * NUM_SUBCORES
B_PER_W = BATCH // NW
RB = 4
NROUND = B_PER_W // RB
NBUF = 4

_mesh = plsc.VectorSubcoreMesh(core_axis_name="c", subcore_axis_name="s")


@functools.partial(
    pl.kernel,
    mesh=_mesh,
    out_type=jax.ShapeDtypeStruct((EMBED_DIM, BATCH), jnp.float32),
    scratch_types=[
        pltpu.VMEM_SHARED((NUM_SUBCORES, B_PER_W), jnp.int32),
        pltpu.SMEM((B_PER_W,), jnp.int32),
        pltpu.VMEM((NBUF, RB, EMBED_DIM, 128), jnp.float32),
        pltpu.VMEM((EMBED_DIM, B_PER_W), jnp.float32),
        [pltpu.SemaphoreType.DMA] * NBUF,
    ],
    compiler_params=pltpu.CompilerParams(
        use_tc_tiling_on_sc=True, needs_layout_passes=False
    ),
)
def _gather_kernel(tv_hbm, idx_hbm, out_hbm, idx_sh, idx_s, blk_v, stage_v,
                   sems):
    sid = lax.axis_index("s")
    wid = sid * NUM_CORES + lax.axis_index("c")
    base = wid * B_PER_W
    pltpu.sync_copy(idx_hbm.at[pl.ds(base, B_PER_W)], idx_sh.at[sid])
    pltpu.sync_copy(idx_sh.at[sid], idx_s)

    d_lo = lax.iota(jnp.int32, LANES)
    d_hi = d_lo + LANES

    def fire(r, buf):
        k0 = r * RB
        for t in range(RB):
            i = idx_s[k0 + t]
            blk = pl.multiple_of((i // 128) * 128, 128)
            pltpu.async_copy(
                tv_hbm.at[:, pl.ds(blk, 128)], blk_v.at[buf, t], sems[buf]
            )

    def drain_extract(r, buf):
        pltpu.make_async_copy(
            tv_hbm.at[:, pl.ds(0, RB * 128)], blk_v.at[buf], sems[buf]
        ).wait()
        k0 = r * RB
        for t in range(RB):
            i = idx_s[k0 + t]
            j = lax.rem(i, 128)
            col = jnp.full((LANES,), j, jnp.int32)
            row_k = jnp.full((LANES,), k0 + t, jnp.int32)
            v_lo = plsc.load_gather(blk_v.at[buf, t], [d_lo, col])
            v_hi = plsc.load_gather(blk_v.at[buf, t], [d_hi, col])
            plsc.store_scatter(stage_v, [d_lo, row_k], v_lo)
            plsc.store_scatter(stage_v, [d_hi, row_k], v_hi)

    for q in range(NBUF - 1):
        fire(q, q)

    def group_body(p, c):
        r0 = p * NBUF
        for q in range(NBUF):
            r = r0 + q
            fb = (q + NBUF - 1) % NBUF

            @pl.when(r + NBUF - 1 < NROUND)
            def _():
                fire(r + NBUF - 1, fb)

            drain_extract(r, q)
        return c

    lax.fori_loop(0, NROUND // NBUF, group_body, 0)
    pltpu.sync_copy(stage_v, out_hbm.at[:, pl.ds(base, B_PER_W)])


def kernel(broadcaster, table):
    idx = broadcaster.astype(jnp.int32)
    out_t = _gather_kernel(table.T, idx)
    return out_t.T
